# baseline (speedup 1.0000x reference)
import jax
import jax.numpy as jnp
from jax import lax
from jax.experimental import pallas as pl
from jax.experimental.pallas import tpu as pltpu
from jax.experimental.pallas import tpu_sc as plsc

NENT = 10000
NSEG = 10240
CH = 128
NEDGE = 320000
NRELROW = 9
NC = 2
NS = 16
NW = NC * NS
EPW = 10240
EPAD = NW * EPW
K = 80
KH = K // 2
NCHUNK = EPW // K
NG = K // 16
BLK = 8
BLKE = BLK * K
NBLOCK = NCHUNK // BLK
DEPTH = 4
ROWS_PER_TILE = NSEG // NS
K2 = 64
NCHUNK2 = EPW // K2
BLK2 = 8
NBLOCK2 = NCHUNK2 // BLK2
MBLK = 128
MROUNDS = NSEG // MBLK

_MESH = plsc.VectorSubcoreMesh(core_axis_name="c", subcore_axis_name="s")
_f32 = jnp.float32
_i32 = jnp.int32


def _c(v):
    return jnp.array(v, _i32)


def _worker_id():
    return lax.axis_index("s") * _c(NC) + lax.axis_index("c")


def _retry_scatter_max(tab, hidx, sv):
    def bdy(go):
        cur = plsc.load_gather(tab, [hidx])
        plsc.store_scatter(tab, [hidx], jnp.maximum(sv, cur), mask=sv > cur)
        chk = plsc.load_gather(tab, [hidx])
        return jnp.max((sv > chk).astype(_i32))
    lax.while_loop(lambda go: go > _c(0), bdy, _c(1))


def _gather_rows_split(table_hbm, idxb, rows, sem):
    return (pltpu.async_copy(table_hbm.at[idxb], rows, sem),)


def _sca_body(x_hbm, rel_emb_hbm, head_hbm, rel_hbm, tail_hbm,
              scores_hbm, maxpart_hbm,
              headblk, relblk, tailblk, sball, maxtab, reltab,
              arows0, arows1, arows2, arows3,
              trows0, trows1, trows2, trows3,
              cidxb0, cidxb1, cidxb2, cidxb3,
              tailb0, tailb1, tailb2, tailb3,
              sem0, sem1, sem2, sem3):
    w = _worker_id()
    base0 = w * _c(EPW)
    lanes = lax.iota(_i32, 16)
    zero16 = jnp.zeros((16,), _f32)
    zero16i = jnp.zeros((16,), _i32)

    pltpu.sync_copy(rel_emb_hbm, reltab)

    def initb(i, carry):
        plsc.store_scatter(maxtab, [lanes + i * _c(16)], zero16)
        return carry
    lax.fori_loop(_c(0), _c(NSEG // 16), initb, _c(0))

    arows = (arows0, arows1, arows2, arows3)
    trows = (trows0, trows1, trows2, trows3)
    cidxb = (cidxb0, cidxb1, cidxb2, cidxb3)
    tailb = (tailb0, tailb1, tailb2, tailb3)
    sems = (sem0, sem1, sem2, sem3)

    def issue(j, slot):
        for g in range(NG):
            off = j * K + g * 16
            cidxb[slot][pl.ds(g * 16, 16)] = headblk[pl.ds(off, 16)]
            tailb[slot][pl.ds(g * 16, 16)] = tailblk[pl.ds(off, 16)]
        da = _gather_rows_split(x_hbm, cidxb[slot], arows[slot], sems[slot])
        dt = _gather_rows_split(x_hbm, tailb[slot], trows[slot], sems[slot])
        return da + dt

    def block(b, carry):
        eb0 = base0 + b * _c(BLKE)
        pltpu.sync_copy(head_hbm.at[pl.ds(eb0, BLKE)], headblk)
        pltpu.sync_copy(rel_hbm.at[pl.ds(eb0, BLKE)], relblk)
        pltpu.sync_copy(tail_hbm.at[pl.ds(eb0, BLKE)], tailblk)
        ds_ = [None] * BLK
        for p in range(DEPTH):
            ds_[p] = issue(p, p)
        for j in range(BLK):
            sl = j % DEPTH
            for d in ds_[j]:
                d.wait()
            for g in range(NG):
                eidx = lanes + _c(g * 16)
                r16 = relblk[pl.ds(j * K + g * 16, 16)]
                rid = jnp.where(r16 == _c(0), _c(NRELROW - 1), r16 - _c(1))

                def cbody(i2, accs, sl=sl, eidx=eidx, rid=rid):
                    ch0 = i2 * _c(8)
                    outs = list(accs)
                    for k in range(8):
                        col = zero16i + (ch0 + _c(k))
                        av = plsc.load_gather(arows[sl], [eidx, col])
                        tv = plsc.load_gather(trows[sl], [eidx, col])
                        rv = plsc.load_gather(reltab, [rid, col])
                        outs[k % 4] = outs[k % 4] + av * rv * tv
                    return tuple(outs)
                accs = lax.fori_loop(_c(0), _c(CH // 8), cbody,
                                     (zero16, zero16, zero16, zero16))
                dot = (accs[0] + accs[1]) + (accs[2] + accs[3])
                sv = jnp.exp(dot)
                gidx = lanes + b * _c(BLKE) + _c(j * K + g * 16)
                plsc.store_scatter(sball, [gidx], sv)
                hidx = headblk[pl.ds(j * K + g * 16, 16)]
                _retry_scatter_max(maxtab, hidx, sv)
            if j + DEPTH < BLK:
                ds_[j + DEPTH] = issue(j + DEPTH, sl)
        return carry
    lax.fori_loop(_c(0), _c(NBLOCK), block, _c(0))

    pltpu.sync_copy(sball, scores_hbm.at[pl.ds(base0, EPW)])
    pltpu.sync_copy(maxtab, maxpart_hbm.at[w])


_sca = pl.kernel(
    _sca_body,
    out_type=[jax.ShapeDtypeStruct((EPAD,), _f32),
              jax.ShapeDtypeStruct((NW, NSEG), _f32)],
    mesh=_MESH,
    compiler_params=pltpu.CompilerParams(needs_layout_passes=False),
    scratch_types=(
        [pltpu.VMEM((BLKE,), _i32)] * 3
        + [pltpu.VMEM((EPW,), _f32),
           pltpu.VMEM((NSEG,), _f32),
           pltpu.VMEM((NRELROW, CH), _f32)]
        + [pltpu.VMEM((K, CH), _f32)] * 8
        + [pltpu.VMEM((K,), _i32)] * 8
        + [pltpu.SemaphoreType.DMA] * 4
    ),
)


def _scm_body(head_hbm, scores_hbm, maxpart_hbm,
              evals_hbm,
              headall, sball, eall, mtab, mstage):
    w = _worker_id()
    base0 = w * _c(EPW)
    lanes = lax.iota(_i32, 16)

    pltpu.sync_copy(head_hbm.at[pl.ds(base0, EPW)], headall)
    pltpu.sync_copy(scores_hbm.at[pl.ds(base0, EPW)], sball)

    def mround(r, carry):
        seg0 = r * _c(MBLK)
        pltpu.sync_copy(maxpart_hbm.at[:, pl.ds(seg0, MBLK)], mstage)
        for g in range(MBLK // 16):
            idx = lanes + _c(g * 16)
            m = plsc.load_gather(mstage, [jnp.full((16,), 0, _i32), idx])
            for j in range(1, NW):
                vj = plsc.load_gather(mstage, [jnp.full((16,), j, _i32), idx])
                m = jnp.maximum(m, vj)
            plsc.store_scatter(mtab, [idx + seg0], m)
        return carry
    lax.fori_loop(_c(0), _c(MROUNDS), mround, _c(0))

    def ebody(i, carry):
        idx = lanes + i * _c(16)
        h16 = plsc.load_gather(headall, [idx])
        m16 = plsc.load_gather(mtab, [h16])
        s16 = plsc.load_gather(sball, [idx])
        plsc.store_scatter(eall, [idx], jnp.exp(s16 - m16))
        return carry
    lax.fori_loop(_c(0), _c(EPW // 16), ebody, _c(0))

    pltpu.sync_copy(eall, evals_hbm.at[pl.ds(base0, EPW)])


_scm = pl.kernel(
    _scm_body,
    out_type=[jax.ShapeDtypeStruct((EPAD,), _f32)],
    mesh=_MESH,
    compiler_params=pltpu.CompilerParams(needs_layout_passes=False),
    scratch_types=[
        pltpu.VMEM((EPW,), _i32),
        pltpu.VMEM((EPW,), _f32),
        pltpu.VMEM((EPW,), _f32),
        pltpu.VMEM((NSEG,), _f32),
        pltpu.VMEM((NW, MBLK), _f32),
    ],
)


def _zero_rows_buf(buf, nrows):
    lanes = lax.iota(_i32, 16)
    zero16 = jnp.zeros((16,), _f32)

    def zb(i, carry):
        flat = lanes + i * _c(16)
        plsc.store_scatter(buf, [lax.shift_right_logical(flat, _c(7)),
                                 lax.bitwise_and(flat, _c(127))], zero16)
        return carry
    lax.fori_loop(_c(0), _c(nrows * CH // 16), zb, _c(0))


def _zero_spmem_slice(spmem, buf, sid, nrows):
    def zloop(j, carry):
        pltpu.sync_copy(
            buf,
            spmem.at[pl.ds(sid * _c(ROWS_PER_TILE) + j * _c(nrows), nrows)])
        return carry
    lax.fori_loop(_c(0), _c(ROWS_PER_TILE // nrows), zloop, _c(0))


def _dump_spmem(spmem, buf, ypart_hbm, c, sid, nrows):
    for j in range(ROWS_PER_TILE // nrows):
        row = sid * _c(ROWS_PER_TILE) + _c(j * nrows)
        pltpu.sync_copy(spmem.at[pl.ds(row, nrows)], buf)
        pltpu.sync_copy(buf, ypart_hbm.at[c, pl.ds(row, nrows)])


def _scagg_body(x_hbm, idx_hbm, tail_hbm, ev_hbm,
                ypart_hbm,
                idxiblk, idxtblk, evblk,
                idxb0, idxb1, idxb2, idxb3,
                rows0, rows1, rows2, rows3,
                spmem, sem0, sem1, sem2, sem3):
    c = lax.axis_index("c")
    sid = lax.axis_index("s")
    w = _worker_id()
    base0 = w * _c(EPW)
    lanes = lax.iota(_i32, 16)

    _zero_rows_buf(rows0, K)
    _zero_spmem_slice(spmem, rows0, sid, K)
    plsc.subcore_barrier()

    rows = (rows0, rows1, rows2, rows3)
    idxb = (idxb0, idxb1, idxb2, idxb3)
    sems = (sem0, sem1, sem2, sem3)

    def block(b, carry):
        eb0 = base0 + b * _c(BLKE)
        pltpu.sync_copy(idx_hbm.at[pl.ds(eb0, BLKE)], idxiblk)
        pltpu.sync_copy(tail_hbm.at[pl.ds(eb0, BLKE)], idxtblk)
        pltpu.sync_copy(ev_hbm.at[pl.ds(eb0, BLKE)], evblk)
        ds_ = [None] * BLK

        def issue(j, slot):
            return _gather_rows_split(
                x_hbm, idxtblk.at[pl.ds(j * K, K)], rows[slot], sems[slot])
        for p in range(DEPTH):
            ds_[p] = issue(p, p)
        for j in range(BLK):
            sl = j % DEPTH
            for d in ds_[j]:
                d.wait()
            for g in range(NG):
                iv = idxiblk[pl.ds(j * K + g * 16, 16)]
                idxb[sl][pl.ds(g * 16, 16)] = iv
                ev = evblk[pl.ds(j * K + g * 16, 16)]
                eidx = lanes + _c(g * 16)

                def sbody(i2, carry2, sl=sl, eidx=eidx, ev=ev):
                    ch0 = i2 * _c(8)
                    for k in range(8):
                        col = jnp.zeros((16,), _i32) + (ch0 + _c(k))
                        tv = plsc.load_gather(rows[sl], [eidx, col])
                        plsc.store_scatter(rows[sl], [eidx, col], tv * ev)
                    return carry2
                lax.fori_loop(_c(0), _c(CH // 8), sbody, _c(0))
            pltpu.sync_copy(rows[sl], spmem.at[idxb[sl]], add=True)
            if j + DEPTH < BLK:
                ds_[j + DEPTH] = issue(j + DEPTH, sl)
        return carry
    lax.fori_loop(_c(0), _c(NBLOCK), block, _c(0))

    plsc.subcore_barrier()
    _dump_spmem(spmem, rows0, ypart_hbm, c, sid, K)


_scagg = pl.kernel(
    _scagg_body,
    out_type=[jax.ShapeDtypeStruct((NC, NSEG, CH), _f32)],
    mesh=_MESH,
    compiler_params=pltpu.CompilerParams(needs_layout_passes=False),
    scratch_types=(
        [pltpu.VMEM((BLKE,), _i32)] * 2
        + [pltpu.VMEM((BLKE,), _f32)]
        + [pltpu.VMEM((K,), _i32)] * 4
        + [pltpu.VMEM((K, CH), _f32)] * 4
        + [pltpu.VMEM_SHARED((NSEG, CH), _f32)]
        + [pltpu.SemaphoreType.DMA] * 4
    ),
)


def _scf_body(x_hbm, rel_emb_hbm, head_hbm, rel_hbm, tail_hbm,
              ypart_hbm,
              hblk, rblk, tblk, reltab, headbf,
              hrows0, hrows1, trows0, trows1, orows,
              hb0, hb1, tb0, tb1,
              spmem, sem0, sem1):
    c = lax.axis_index("c")
    sid = lax.axis_index("s")
    w = _worker_id()
    base0 = w * _c(EPW)
    lanes = lax.iota(_i32, 16)
    zero16 = jnp.zeros((16,), _f32)
    zero16i = jnp.zeros((16,), _i32)

    _zero_rows_buf(orows, K2)
    _zero_spmem_slice(spmem, orows, sid, K2)
    plsc.subcore_barrier()
    pltpu.sync_copy(rel_emb_hbm, reltab)

    hrows = (hrows0, hrows1)
    trows = (trows0, trows1)
    hb = (hb0, hb1)
    tb = (tb0, tb1)
    sems = (sem0, sem1)

    def issue(j, slot):
        for g in range(K2 // 16):
            off = j * K2 + g * 16
            hb[slot][pl.ds(g * 16, 16)] = hblk[pl.ds(off, 16)]
            tb[slot][pl.ds(g * 16, 16)] = tblk[pl.ds(off, 16)]
        dh = pltpu.async_copy(x_hbm.at[hb[slot]], hrows[slot], sems[slot])
        dt = pltpu.async_copy(x_hbm.at[tb[slot]], trows[slot], sems[slot])
        return (dh, dt)

    def block(b, carry):
        eb0 = base0 + b * _c(BLK2 * K2)
        pltpu.sync_copy(head_hbm.at[pl.ds(eb0, BLK2 * K2)], hblk)
        pltpu.sync_copy(rel_hbm.at[pl.ds(eb0, BLK2 * K2)], rblk)
        pltpu.sync_copy(tail_hbm.at[pl.ds(eb0, BLK2 * K2)], tblk)
        ds_ = [None] * BLK2
        for p in range(2):
            ds_[p] = issue(p, p)
        for j in range(BLK2):
            sl = j % 2
            for d in ds_[j]:
                d.wait()
            for g in range(K2 // 16):
                off = j * K2 + g * 16
                eidx = lanes + _c(g * 16)
                headbf[pl.ds(g * 16, 16)] = hblk[pl.ds(off, 16)]
                r16 = rblk[pl.ds(off, 16)]
                rid = jnp.where(r16 == _c(0), _c(NRELROW - 1), r16 - _c(1))

                def cbody(i2, accs, sl=sl, eidx=eidx, rid=rid):
                    ch0 = i2 * _c(8)
                    outs = list(accs)
                    for k in range(8):
                        col = zero16i + (ch0 + _c(k))
                        hv = plsc.load_gather(hrows[sl], [eidx, col])
                        tv = plsc.load_gather(trows[sl], [eidx, col])
                        rv = plsc.load_gather(reltab, [rid, col])
                        outs[k % 4] = outs[k % 4] + hv * rv * tv
                    return tuple(outs)
                accs = lax.fori_loop(_c(0), _c(CH // 8), cbody,
                                     (zero16, zero16, zero16, zero16))
                ev = jnp.exp(jnp.exp((accs[0] + accs[1]) + (accs[2] + accs[3])))

                def sbody(i2, carry2, sl=sl, eidx=eidx, ev=ev):
                    ch0 = i2 * _c(8)
                    for k in range(8):
                        col = jnp.zeros((16,), _i32) + (ch0 + _c(k))
                        tv = plsc.load_gather(trows[sl], [eidx, col])
                        plsc.store_scatter(orows, [eidx, col], tv * ev)
                    return carry2
                lax.fori_loop(_c(0), _c(CH // 8), sbody, _c(0))
            if j + 2 < BLK2:
                ds_[j + 2] = issue(j + 2, sl)
            pltpu.sync_copy(orows, spmem.at[headbf], add=True)
        return carry
    lax.fori_loop(_c(0), _c(NBLOCK2), block, _c(0))

    plsc.subcore_barrier()
    _dump_spmem(spmem, orows, ypart_hbm, c, sid, K2)


_scf = pl.kernel(
    _scf_body,
    out_type=[jax.ShapeDtypeStruct((NC, NSEG, CH), _f32)],
    mesh=_MESH,
    compiler_params=pltpu.CompilerParams(needs_layout_passes=False),
    scratch_types=(
        [pltpu.VMEM((BLK2 * K2,), _i32)] * 3
        + [pltpu.VMEM((NRELROW, CH), _f32)]
        + [pltpu.VMEM((K2,), _i32)]
        + [pltpu.VMEM((K2, CH), _f32)] * 5
        + [pltpu.VMEM((K2,), _i32)] * 4
        + [pltpu.VMEM_SHARED((NSEG, CH), _f32)]
        + [pltpu.SemaphoreType.DMA] * 2
    ),
)


_RB = 1280


def _z(v=0):
    return jnp.array(v, _i32)


def _norm_rows(a):
    ss = jnp.sum(a * a, axis=1, keepdims=True)
    return a * lax.rsqrt(jnp.maximum(ss, 1e-24))


def _tc_merge_body(pp_ref, x_ref):
    x_ref[...] = _norm_rows(pp_ref[0] + pp_ref[1])


_tc_merge = pl.pallas_call(
    _tc_merge_body,
    grid=(NSEG // _RB,),
    in_specs=[pl.BlockSpec((NC, _RB, CH), lambda b: (_z(), b, _z()))],
    out_specs=pl.BlockSpec((_RB, CH), lambda b: (b, _z())),
    out_shape=jax.ShapeDtypeStruct((NSEG, CH), _f32),
)


_EXTRA = EPAD - NEDGE


def kernel(user_emb, item_emb, edge_index, edge_type, inter_edge,
           inter_edge_w, relation_emb):
    del user_emb
    head = jnp.pad(edge_index[0].astype(_i32), (0, _EXTRA),
                   constant_values=NSEG - 1)
    tail = jnp.pad(edge_index[1].astype(_i32), (0, _EXTRA))
    rel = jnp.pad(edge_type.astype(_i32), (0, _EXTRA), constant_values=1)
    src = jnp.pad(inter_edge[0].astype(_i32), (0, _EXTRA),
                  constant_values=NSEG - 1)
    dst = jnp.pad(inter_edge[1].astype(_i32), (0, _EXTRA))
    iw = jnp.pad(inter_edge_w.astype(_f32), (0, _EXTRA))
    relemb = relation_emb.astype(_f32)

    x = jnp.pad(item_emb.astype(_f32), ((0, NSEG - NENT), (0, 0)))
    scores, maxpart = _sca(x, relemb, head, rel, tail)
    (evals,) = _scm(head, scores, maxpart)
    (ypart,) = _scagg(x, head, tail, evals)
    x = _tc_merge(ypart)
    (ypart2,) = _scf(x, relemb, head, rel, tail)
    x = _tc_merge(ypart2)
    (upart,) = _scagg(x, src, dst, iw)
    user_out = _tc_merge(upart)
    return user_out[:NENT], x[:NENT]


# speedup vs baseline: 1.2052x; 1.2052x over previous
import jax
import jax.numpy as jnp
from jax import lax
from jax.experimental import pallas as pl
from jax.experimental.pallas import tpu as pltpu
from jax.experimental.pallas import tpu_sc as plsc

NENT = 10000
NSEG = 10240
CH = 128
NEDGE = 320000
NRELROW = 9
NC = 2
NS = 16
NW = NC * NS
EPW = NEDGE // NW
K = 80
NCHUNK = EPW // K
NG = K // 16
BLK = 5
BLKE = BLK * K
NBLOCK = NCHUNK // BLK
ROWS_PER_TILE = NSEG // NS
MBLK = 128
MROUNDS = NSEG // MBLK

_MESH = plsc.VectorSubcoreMesh(core_axis_name="c", subcore_axis_name="s")
_f32 = jnp.float32
_i32 = jnp.int32


def _c(v):
    return jnp.array(v, _i32)


def _worker_id():
    return lax.axis_index("s") * _c(NC) + lax.axis_index("c")


def _retry_scatter_max(tab, hidx, sv):
    def bdy(go):
        cur = plsc.load_gather(tab, [hidx])
        plsc.store_scatter(tab, [hidx], jnp.maximum(sv, cur), mask=sv > cur)
        chk = plsc.load_gather(tab, [hidx])
        return jnp.max((sv > chk).astype(_i32))
    lax.while_loop(lambda go: go > _c(0), bdy, _c(1))


def _sca_body(a_hbm, x_hbm, head_hbm, rel_hbm, tail_hbm,
              scores_hbm, maxpart_hbm,
              headall, relall, tailall, cidxall, sball, maxtab,
              arows0, arows1, trows0, trows1, cidxb0, cidxb1, tailb0, tailb1,
              sa0, sa1, st0, st1):
    w = _worker_id()
    base0 = w * _c(EPW)
    lanes = lax.iota(_i32, 16)
    zero16 = jnp.zeros((16,), _f32)
    zero16i = jnp.zeros((16,), _i32)

    pltpu.sync_copy(head_hbm.at[pl.ds(base0, EPW)], headall)
    pltpu.sync_copy(rel_hbm.at[pl.ds(base0, EPW)], relall)
    pltpu.sync_copy(tail_hbm.at[pl.ds(base0, EPW)], tailall)

    def initb(i, carry):
        plsc.store_scatter(maxtab, [lanes + i * _c(16)], zero16)
        return carry
    lax.fori_loop(_c(0), _c(NSEG // 16), initb, _c(0))

    def cidx_build(i, carry):
        idx = lanes + i * _c(16)
        h16 = plsc.load_gather(headall, [idx])
        r16 = plsc.load_gather(relall, [idx])
        rid = jnp.where(r16 == _c(0), _c(NRELROW - 1), r16 - _c(1))
        plsc.store_scatter(cidxall, [idx], rid * _c(NSEG) + h16)
        return carry
    lax.fori_loop(_c(0), _c(EPW // 16), cidx_build, _c(0))

    arows = (arows0, arows1)
    trows = (trows0, trows1)
    cidxb = (cidxb0, cidxb1)
    tailb = (tailb0, tailb1)
    sa = (sa0, sa1)
    st = (st0, st1)

    def issue(ci, jmod):
        base_l = ci * _c(K)
        for g in range(NG):
            gidx = lanes + base_l + _c(g * 16)
            cidxb[jmod][pl.ds(g * 16, 16)] = plsc.load_gather(cidxall, [gidx])
            tailb[jmod][pl.ds(g * 16, 16)] = plsc.load_gather(tailall, [gidx])
        da = pltpu.async_copy(a_hbm.at[cidxb[jmod]], arows[jmod], sa[jmod])
        dt = pltpu.async_copy(x_hbm.at[tailb[jmod]], trows[jmod], st[jmod])
        return da, dt

    def block(b, carry):
        ci0 = b * _c(BLK)
        ds_ = [None] * BLK
        ds_[0] = issue(ci0, 0)
        for j in range(BLK):
            jm = j % 2
            if j + 1 < BLK:
                ds_[j + 1] = issue(ci0 + _c(j + 1), (j + 1) % 2)
            ds_[j][0].wait()
            ds_[j][1].wait()
            base_l = (ci0 + _c(j)) * _c(K)
            for g in range(NG):
                eidx = lanes + _c(g * 16)

                def cbody(i2, accs, jm=jm, eidx=eidx):
                    ch0 = i2 * _c(16)
                    outs = list(accs)
                    for k in range(16):
                        col = zero16i + (ch0 + _c(k))
                        av = plsc.load_gather(arows[jm], [eidx, col])
                        tv = plsc.load_gather(trows[jm], [eidx, col])
                        outs[k % 4] = outs[k % 4] + av * tv
                    return tuple(outs)
                accs = lax.fori_loop(_c(0), _c(CH // 16), cbody,
                                     (zero16, zero16, zero16, zero16))
                dot = (accs[0] + accs[1]) + (accs[2] + accs[3])
                sv = jnp.exp(dot)
                gidx = lanes + base_l + _c(g * 16)
                plsc.store_scatter(sball, [gidx], sv)
                hidx = plsc.load_gather(headall, [gidx])
                _retry_scatter_max(maxtab, hidx, sv)
        return carry
    lax.fori_loop(_c(0), _c(NBLOCK), block, _c(0))

    pltpu.sync_copy(sball, scores_hbm.at[pl.ds(base0, EPW)])
    pltpu.sync_copy(maxtab, maxpart_hbm.at[w])


_sca = pl.kernel(
    _sca_body,
    out_type=[jax.ShapeDtypeStruct((NEDGE,), _f32),
              jax.ShapeDtypeStruct((NW, NSEG), _f32)],
    mesh=_MESH,
    compiler_params=pltpu.CompilerParams(needs_layout_passes=False),
    scratch_types=[
        pltpu.VMEM((EPW,), _i32),
        pltpu.VMEM((EPW,), _i32),
        pltpu.VMEM((EPW,), _i32),
        pltpu.VMEM((EPW,), _i32),
        pltpu.VMEM((EPW,), _f32),
        pltpu.VMEM((NSEG,), _f32),
        pltpu.VMEM((K, CH), _f32),
        pltpu.VMEM((K, CH), _f32),
        pltpu.VMEM((K, CH), _f32),
        pltpu.VMEM((K, CH), _f32),
        pltpu.VMEM((K,), _i32),
        pltpu.VMEM((K,), _i32),
        pltpu.VMEM((K,), _i32),
        pltpu.VMEM((K,), _i32),
        pltpu.SemaphoreType.DMA,
        pltpu.SemaphoreType.DMA,
        pltpu.SemaphoreType.DMA,
        pltpu.SemaphoreType.DMA,
    ],
)


def _zero_rows_buf(buf):
    lanes = lax.iota(_i32, 16)
    zero16 = jnp.zeros((16,), _f32)

    def zb(i, carry):
        flat = lanes + i * _c(16)
        plsc.store_scatter(buf, [lax.shift_right_logical(flat, _c(7)),
                                 lax.bitwise_and(flat, _c(127))], zero16)
        return carry
    lax.fori_loop(_c(0), _c(K * CH // 16), zb, _c(0))


def _zero_spmem_slice(spmem, buf, sid):
    def zloop(j, carry):
        pltpu.sync_copy(
            buf, spmem.at[pl.ds(sid * _c(ROWS_PER_TILE) + j * _c(K), K)])
        return carry
    lax.fori_loop(_c(0), _c(ROWS_PER_TILE // K), zloop, _c(0))


def _dump_spmem(spmem, buf, ypart_hbm, c, sid):
    for j in range(ROWS_PER_TILE // K):
        row = sid * _c(ROWS_PER_TILE) + _c(j * K)
        pltpu.sync_copy(spmem.at[pl.ds(row, K)], buf)
        pltpu.sync_copy(buf, ypart_hbm.at[c, pl.ds(row, K)])


def _scb_body(x_hbm, head_hbm, tail_hbm, scores_hbm, maxpart_hbm,
              ypart_hbm,
              idxhblk, idxtblk, sblk, headb0, headb1, trows0, trows1,
              mtab, mstage, spmem, st0, st1):
    c = lax.axis_index("c")
    sid = lax.axis_index("s")
    w = _worker_id()
    base0 = w * _c(EPW)
    lanes = lax.iota(_i32, 16)

    _zero_rows_buf(trows0)
    _zero_spmem_slice(spmem, trows0, sid)
    plsc.subcore_barrier()

    def mround(r, carry):
        seg0 = r * _c(MBLK)
        pltpu.sync_copy(maxpart_hbm.at[:, pl.ds(seg0, MBLK)], mstage)
        for g in range(MBLK // 16):
            idx = lanes + _c(g * 16)
            m = plsc.load_gather(mstage, [jnp.full((16,), 0, _i32), idx])
            for j in range(1, NW):
                vj = plsc.load_gather(mstage, [jnp.full((16,), j, _i32), idx])
                m = jnp.maximum(m, vj)
            plsc.store_scatter(mtab, [idx + seg0], m)
        return carry
    lax.fori_loop(_c(0), _c(MROUNDS), mround, _c(0))

    trows = (trows0, trows1)
    headb = (headb0, headb1)
    st = (st0, st1)

    def block(b, carry):
        eb0 = base0 + b * _c(BLKE)
        pltpu.sync_copy(head_hbm.at[pl.ds(eb0, BLKE)], idxhblk)
        pltpu.sync_copy(tail_hbm.at[pl.ds(eb0, BLKE)], idxtblk)
        pltpu.sync_copy(scores_hbm.at[pl.ds(eb0, BLKE)], sblk)
        ds_ = [None] * BLK
        ds_[0] = pltpu.async_copy(x_hbm.at[idxtblk.at[pl.ds(0, K)]],
                                  trows0, st0)
        for j in range(BLK):
            jm = j % 2
            if j + 1 < BLK:
                ds_[j + 1] = pltpu.async_copy(
                    x_hbm.at[idxtblk.at[pl.ds((j + 1) * K, K)]],
                    trows[(j + 1) % 2], st[(j + 1) % 2])
            ds_[j].wait()
            for g in range(NG):
                hv = idxhblk[pl.ds(j * K + g * 16, 16)]
                headb[jm][pl.ds(g * 16, 16)] = hv
                m16 = plsc.load_gather(mtab, [hv])
                ev = jnp.exp(sblk[pl.ds(j * K + g * 16, 16)] - m16)
                eidx = lanes + _c(g * 16)

                def sbody(i2, carry2, jm=jm, eidx=eidx, ev=ev):
                    ch0 = i2 * _c(16)
                    for k in range(16):
                        col = jnp.zeros((16,), _i32) + (ch0 + _c(k))
                        tv = plsc.load_gather(trows[jm], [eidx, col])
                        plsc.store_scatter(trows[jm], [eidx, col], tv * ev)
                    return carry2
                lax.fori_loop(_c(0), _c(CH // 16), sbody, _c(0))
            pltpu.sync_copy(trows[jm], spmem.at[headb[jm]], add=True)
        return carry
    lax.fori_loop(_c(0), _c(NBLOCK), block, _c(0))

    plsc.subcore_barrier()
    _dump_spmem(spmem, trows0, ypart_hbm, c, sid)


_scb = pl.kernel(
    _scb_body,
    out_type=[jax.ShapeDtypeStruct((NC, NSEG, CH), _f32)],
    mesh=_MESH,
    compiler_params=pltpu.CompilerParams(needs_layout_passes=False),
    scratch_types=[
        pltpu.VMEM((BLKE,), _i32),
        pltpu.VMEM((BLKE,), _i32),
        pltpu.VMEM((BLKE,), _f32),
        pltpu.VMEM((K,), _i32),
        pltpu.VMEM((K,), _i32),
        pltpu.VMEM((K, CH), _f32),
        pltpu.VMEM((K, CH), _f32),
        pltpu.VMEM((NSEG,), _f32),
        pltpu.VMEM((NW, MBLK), _f32),
        pltpu.VMEM_SHARED((NSEG, CH), _f32),
        pltpu.SemaphoreType.DMA,
        pltpu.SemaphoreType.DMA,
    ],
)


def _scu_body(x_hbm, src_hbm, dst_hbm, w_hbm,
              upart_hbm,
              idxsblk, idxdblk, wblk, srcb0, srcb1, xrows0, xrows1,
              spmem, st0, st1):
    c = lax.axis_index("c")
    sid = lax.axis_index("s")
    w = _worker_id()
    base0 = w * _c(EPW)
    lanes = lax.iota(_i32, 16)

    _zero_rows_buf(xrows0)
    _zero_spmem_slice(spmem, xrows0, sid)
    plsc.subcore_barrier()

    xrows = (xrows0, xrows1)
    srcb = (srcb0, srcb1)
    st = (st0, st1)

    def block(b, carry):
        eb0 = base0 + b * _c(BLKE)
        pltpu.sync_copy(src_hbm.at[pl.ds(eb0, BLKE)], idxsblk)
        pltpu.sync_copy(dst_hbm.at[pl.ds(eb0, BLKE)], idxdblk)
        pltpu.sync_copy(w_hbm.at[pl.ds(eb0, BLKE)], wblk)
        ds_ = [None] * BLK
        ds_[0] = pltpu.async_copy(x_hbm.at[idxdblk.at[pl.ds(0, K)]],
                                  xrows0, st0)
        for j in range(BLK):
            jm = j % 2
            if j + 1 < BLK:
                ds_[j + 1] = pltpu.async_copy(
                    x_hbm.at[idxdblk.at[pl.ds((j + 1) * K, K)]],
                    xrows[(j + 1) % 2], st[(j + 1) % 2])
            ds_[j].wait()
            for g in range(NG):
                sv = idxsblk[pl.ds(j * K + g * 16, 16)]
                srcb[jm][pl.ds(g * 16, 16)] = sv
                ev = wblk[pl.ds(j * K + g * 16, 16)]
                eidx = lanes + _c(g * 16)

                def sbody(i2, carry2, jm=jm, eidx=eidx, ev=ev):
                    ch0 = i2 * _c(16)
                    for k in range(16):
                        col = jnp.zeros((16,), _i32) + (ch0 + _c(k))
                        tv = plsc.load_gather(xrows[jm], [eidx, col])
                        plsc.store_scatter(xrows[jm], [eidx, col], tv * ev)
                    return carry2
                lax.fori_loop(_c(0), _c(CH // 16), sbody, _c(0))
            pltpu.sync_copy(xrows[jm], spmem.at[srcb[jm]], add=True)
        return carry
    lax.fori_loop(_c(0), _c(NBLOCK), block, _c(0))

    plsc.subcore_barrier()
    _dump_spmem(spmem, xrows0, upart_hbm, c, sid)


_scu = pl.kernel(
    _scu_body,
    out_type=[jax.ShapeDtypeStruct((NC, NSEG, CH), _f32)],
    mesh=_MESH,
    compiler_params=pltpu.CompilerParams(needs_layout_passes=False),
    scratch_types=[
        pltpu.VMEM((BLKE,), _i32),
        pltpu.VMEM((BLKE,), _i32),
        pltpu.VMEM((BLKE,), _f32),
        pltpu.VMEM((K,), _i32),
        pltpu.VMEM((K,), _i32),
        pltpu.VMEM((K, CH), _f32),
        pltpu.VMEM((K, CH), _f32),
        pltpu.VMEM_SHARED((NSEG, CH), _f32),
        pltpu.SemaphoreType.DMA,
        pltpu.SemaphoreType.DMA,
    ],
)


_RB = 1280


def _z(v=0):
    return jnp.array(v, _i32)


def _tc_prep_body(x_ref, rel_ref, a_ref):
    r = pl.program_id(1)
    a_ref[...] = x_ref[...] * rel_ref[pl.ds(r, 1), :]


_tc_prep = pl.pallas_call(
    _tc_prep_body,
    grid=(NSEG // _RB, NRELROW),
    in_specs=[pl.BlockSpec((_RB, CH), lambda b, r: (b, _z())),
              pl.BlockSpec((NRELROW, CH), lambda b, r: (_z(), _z())),],
    out_specs=pl.BlockSpec((_RB, CH), lambda b, r: (r * _z(NSEG // _RB) + b, _z())),
    out_shape=jax.ShapeDtypeStruct((NRELROW * NSEG, CH), _f32),
)


def _norm_rows(a):
    ss = jnp.sum(a * a, axis=1, keepdims=True)
    return a * lax.rsqrt(jnp.maximum(ss, 1e-24))


def _tc_merge_prep_body(pp_ref, rel_ref, x_ref, a_ref):
    r = pl.program_id(1)
    y = _norm_rows(pp_ref[0] + pp_ref[1])
    x_ref[...] = y
    a_ref[...] = y * rel_ref[pl.ds(r, 1), :]


_tc_merge_prep = pl.pallas_call(
    _tc_merge_prep_body,
    grid=(NSEG // _RB, NRELROW),
    in_specs=[pl.BlockSpec((NC, _RB, CH), lambda b, r: (_z(), b, _z())),
              pl.BlockSpec((NRELROW, CH), lambda b, r: (_z(), _z())),],
    out_specs=[pl.BlockSpec((_RB, CH), lambda b, r: (b, _z())),
               pl.BlockSpec((_RB, CH), lambda b, r: (r * _z(NSEG // _RB) + b, _z()))],
    out_shape=[jax.ShapeDtypeStruct((NSEG, CH), _f32),
               jax.ShapeDtypeStruct((NRELROW * NSEG, CH), _f32)],
)


def _tc_merge_body(pp_ref, x_ref):
    x_ref[...] = _norm_rows(pp_ref[0] + pp_ref[1])


_tc_merge = pl.pallas_call(
    _tc_merge_body,
    grid=(NSEG // _RB,),
    in_specs=[pl.BlockSpec((NC, _RB, CH), lambda b: (_z(), b, _z()))],
    out_specs=pl.BlockSpec((_RB, CH), lambda b: (b, _z())),
    out_shape=jax.ShapeDtypeStruct((NSEG, CH), _f32),
)


def kernel(user_emb, item_emb, edge_index, edge_type, inter_edge,
           inter_edge_w, relation_emb):
    del user_emb
    head = edge_index[0].astype(_i32)
    tail = edge_index[1].astype(_i32)
    rel = edge_type.astype(_i32)
    src = inter_edge[0].astype(_i32)
    dst = inter_edge[1].astype(_i32)
    iw = inter_edge_w.astype(_f32)
    relemb = relation_emb.astype(_f32)

    x = jnp.pad(item_emb.astype(_f32), ((0, NSEG - NENT), (0, 0)))
    a = _tc_prep(x, relemb)
    for hop in range(2):
        scores, maxpart = _sca(a, x, head, rel, tail)
        (ypart,) = _scb(x, head, tail, scores, maxpart)
        if hop == 0:
            x, a = _tc_merge_prep(ypart, relemb)
        else:
            x = _tc_merge(ypart)
    (upart,) = _scu(x, src, dst, iw)
    user_out = _tc_merge(upart)
    return user_out[:NENT], x[:NENT]


# speedup vs baseline: 1.2230x; 1.0148x over previous
import jax
import jax.numpy as jnp
from jax import lax
from jax.experimental import pallas as pl
from jax.experimental.pallas import tpu as pltpu
from jax.experimental.pallas import tpu_sc as plsc

NENT = 10000
NSEG = 10240
CH = 128
NEDGE = 320000
NRELROW = 9
NC = 2
NS = 16
NW = NC * NS
EPW = NEDGE // NW
K = 80
NCHUNK = EPW // K
NG = K // 16
BLK = 5
BLKE = BLK * K
NBLOCK = NCHUNK // BLK
ROWS_PER_TILE = NSEG // NS
MBLK = 128
MROUNDS = NSEG // MBLK

_MESH = plsc.VectorSubcoreMesh(core_axis_name="c", subcore_axis_name="s")
_f32 = jnp.float32
_i32 = jnp.int32


def _c(v):
    return jnp.array(v, _i32)


def _worker_id():
    return lax.axis_index("s") * _c(NC) + lax.axis_index("c")


def _retry_scatter_max(tab, hidx, sv):
    def bdy(go):
        cur = plsc.load_gather(tab, [hidx])
        plsc.store_scatter(tab, [hidx], jnp.maximum(sv, cur), mask=sv > cur)
        chk = plsc.load_gather(tab, [hidx])
        return jnp.max((sv > chk).astype(_i32))
    lax.while_loop(lambda go: go > _c(0), bdy, _c(1))


def _sca_body(a_hbm, x_hbm, head_hbm, rel_hbm, tail_hbm,
              scores_hbm, maxpart_hbm,
              headall, relall, tailall, cidxall, sball, maxtab,
              arows0, arows1, trows0, trows1, cidxb0, cidxb1, tailb0, tailb1,
              sa0, sa1, st0, st1):
    w = _worker_id()
    base0 = w * _c(EPW)
    lanes = lax.iota(_i32, 16)
    zero16 = jnp.zeros((16,), _f32)
    zero16i = jnp.zeros((16,), _i32)

    pltpu.sync_copy(head_hbm.at[pl.ds(base0, EPW)], headall)
    pltpu.sync_copy(rel_hbm.at[pl.ds(base0, EPW)], relall)
    pltpu.sync_copy(tail_hbm.at[pl.ds(base0, EPW)], tailall)

    def initb(i, carry):
        plsc.store_scatter(maxtab, [lanes + i * _c(16)], zero16)
        return carry
    lax.fori_loop(_c(0), _c(NSEG // 16), initb, _c(0))

    def cidx_build(i, carry):
        idx = lanes + i * _c(16)
        h16 = plsc.load_gather(headall, [idx])
        r16 = plsc.load_gather(relall, [idx])
        rid = jnp.where(r16 == _c(0), _c(NRELROW - 1), r16 - _c(1))
        plsc.store_scatter(cidxall, [idx], rid * _c(NSEG) + h16)
        return carry
    lax.fori_loop(_c(0), _c(EPW // 16), cidx_build, _c(0))

    arows = (arows0, arows1)
    trows = (trows0, trows1)
    cidxb = (cidxb0, cidxb1)
    tailb = (tailb0, tailb1)
    sa = (sa0, sa1)
    st = (st0, st1)

    def issue(ci, jmod):
        base_l = ci * _c(K)
        for g in range(NG):
            gidx = lanes + base_l + _c(g * 16)
            cidxb[jmod][pl.ds(g * 16, 16)] = plsc.load_gather(cidxall, [gidx])
            tailb[jmod][pl.ds(g * 16, 16)] = plsc.load_gather(tailall, [gidx])
        da = pltpu.async_copy(a_hbm.at[cidxb[jmod]], arows[jmod], sa[jmod])
        dt = pltpu.async_copy(x_hbm.at[tailb[jmod]], trows[jmod], st[jmod])
        return da, dt

    def block(b, carry):
        ci0 = b * _c(BLK)
        ds_ = [None] * BLK
        ds_[0] = issue(ci0, 0)
        for j in range(BLK):
            jm = j % 2
            if j + 1 < BLK:
                ds_[j + 1] = issue(ci0 + _c(j + 1), (j + 1) % 2)
            ds_[j][0].wait()
            ds_[j][1].wait()
            base_l = (ci0 + _c(j)) * _c(K)
            for g in range(NG):
                eidx = lanes + _c(g * 16)

                def cbody(i2, accs, jm=jm, eidx=eidx):
                    ch0 = i2 * _c(8)
                    outs = list(accs)
                    for k in range(8):
                        col = zero16i + (ch0 + _c(k))
                        av = plsc.load_gather(arows[jm], [eidx, col])
                        tv = plsc.load_gather(trows[jm], [eidx, col])
                        outs[k % 4] = outs[k % 4] + av * tv
                    return tuple(outs)
                accs = lax.fori_loop(_c(0), _c(CH // 8), cbody,
                                     (zero16, zero16, zero16, zero16))
                dot = (accs[0] + accs[1]) + (accs[2] + accs[3])
                sv = jnp.exp(dot)
                gidx = lanes + base_l + _c(g * 16)
                plsc.store_scatter(sball, [gidx], sv)
                hidx = plsc.load_gather(headall, [gidx])
                _retry_scatter_max(maxtab, hidx, sv)
        return carry
    lax.fori_loop(_c(0), _c(NBLOCK), block, _c(0))

    pltpu.sync_copy(sball, scores_hbm.at[pl.ds(base0, EPW)])
    pltpu.sync_copy(maxtab, maxpart_hbm.at[w])


_sca = pl.kernel(
    _sca_body,
    out_type=[jax.ShapeDtypeStruct((NEDGE,), _f32),
              jax.ShapeDtypeStruct((NW, NSEG), _f32)],
    mesh=_MESH,
    compiler_params=pltpu.CompilerParams(needs_layout_passes=False),
    scratch_types=[
        pltpu.VMEM((EPW,), _i32),
        pltpu.VMEM((EPW,), _i32),
        pltpu.VMEM((EPW,), _i32),
        pltpu.VMEM((EPW,), _i32),
        pltpu.VMEM((EPW,), _f32),
        pltpu.VMEM((NSEG,), _f32),
        pltpu.VMEM((K, CH), _f32),
        pltpu.VMEM((K, CH), _f32),
        pltpu.VMEM((K, CH), _f32),
        pltpu.VMEM((K, CH), _f32),
        pltpu.VMEM((K,), _i32),
        pltpu.VMEM((K,), _i32),
        pltpu.VMEM((K,), _i32),
        pltpu.VMEM((K,), _i32),
        pltpu.SemaphoreType.DMA,
        pltpu.SemaphoreType.DMA,
        pltpu.SemaphoreType.DMA,
        pltpu.SemaphoreType.DMA,
    ],
)


def _zero_rows_buf(buf):
    lanes = lax.iota(_i32, 16)
    zero16 = jnp.zeros((16,), _f32)

    def zb(i, carry):
        flat = lanes + i * _c(16)
        plsc.store_scatter(buf, [lax.shift_right_logical(flat, _c(7)),
                                 lax.bitwise_and(flat, _c(127))], zero16)
        return carry
    lax.fori_loop(_c(0), _c(K * CH // 16), zb, _c(0))


def _zero_spmem_slice(spmem, buf, sid):
    def zloop(j, carry):
        pltpu.sync_copy(
            buf, spmem.at[pl.ds(sid * _c(ROWS_PER_TILE) + j * _c(K), K)])
        return carry
    lax.fori_loop(_c(0), _c(ROWS_PER_TILE // K), zloop, _c(0))


def _dump_spmem(spmem, buf, ypart_hbm, c, sid):
    for j in range(ROWS_PER_TILE // K):
        row = sid * _c(ROWS_PER_TILE) + _c(j * K)
        pltpu.sync_copy(spmem.at[pl.ds(row, K)], buf)
        pltpu.sync_copy(buf, ypart_hbm.at[c, pl.ds(row, K)])


def _scb_body(x_hbm, head_hbm, tail_hbm, scores_hbm, maxpart_hbm,
              ypart_hbm,
              idxhblk, idxtblk, sblk, headb0, headb1, trows0, trows1,
              mtab, mstage, spmem, st0, st1):
    c = lax.axis_index("c")
    sid = lax.axis_index("s")
    w = _worker_id()
    base0 = w * _c(EPW)
    lanes = lax.iota(_i32, 16)

    _zero_rows_buf(trows0)
    _zero_spmem_slice(spmem, trows0, sid)
    plsc.subcore_barrier()

    def mround(r, carry):
        seg0 = r * _c(MBLK)
        pltpu.sync_copy(maxpart_hbm.at[:, pl.ds(seg0, MBLK)], mstage)
        for g in range(MBLK // 16):
            idx = lanes + _c(g * 16)
            m = plsc.load_gather(mstage, [jnp.full((16,), 0, _i32), idx])
            for j in range(1, NW):
                vj = plsc.load_gather(mstage, [jnp.full((16,), j, _i32), idx])
                m = jnp.maximum(m, vj)
            plsc.store_scatter(mtab, [idx + seg0], m)
        return carry
    lax.fori_loop(_c(0), _c(MROUNDS), mround, _c(0))

    trows = (trows0, trows1)
    headb = (headb0, headb1)
    st = (st0, st1)

    def block(b, carry):
        eb0 = base0 + b * _c(BLKE)
        pltpu.sync_copy(head_hbm.at[pl.ds(eb0, BLKE)], idxhblk)
        pltpu.sync_copy(tail_hbm.at[pl.ds(eb0, BLKE)], idxtblk)
        pltpu.sync_copy(scores_hbm.at[pl.ds(eb0, BLKE)], sblk)
        ds_ = [None] * BLK
        ds_[0] = pltpu.async_copy(x_hbm.at[idxtblk.at[pl.ds(0, K)]],
                                  trows0, st0)
        for j in range(BLK):
            jm = j % 2
            if j + 1 < BLK:
                ds_[j + 1] = pltpu.async_copy(
                    x_hbm.at[idxtblk.at[pl.ds((j + 1) * K, K)]],
                    trows[(j + 1) % 2], st[(j + 1) % 2])
            ds_[j].wait()
            for g in range(NG):
                hv = idxhblk[pl.ds(j * K + g * 16, 16)]
                headb[jm][pl.ds(g * 16, 16)] = hv
                m16 = plsc.load_gather(mtab, [hv])
                ev = jnp.exp(sblk[pl.ds(j * K + g * 16, 16)] - m16)
                eidx = lanes + _c(g * 16)

                def sbody(i2, carry2, jm=jm, eidx=eidx, ev=ev):
                    ch0 = i2 * _c(8)
                    for k in range(8):
                        col = jnp.zeros((16,), _i32) + (ch0 + _c(k))
                        tv = plsc.load_gather(trows[jm], [eidx, col])
                        plsc.store_scatter(trows[jm], [eidx, col], tv * ev)
                    return carry2
                lax.fori_loop(_c(0), _c(CH // 8), sbody, _c(0))
            pltpu.sync_copy(trows[jm], spmem.at[headb[jm]], add=True)
        return carry
    lax.fori_loop(_c(0), _c(NBLOCK), block, _c(0))

    plsc.subcore_barrier()
    _dump_spmem(spmem, trows0, ypart_hbm, c, sid)


_scb = pl.kernel(
    _scb_body,
    out_type=[jax.ShapeDtypeStruct((NC, NSEG, CH), _f32)],
    mesh=_MESH,
    compiler_params=pltpu.CompilerParams(needs_layout_passes=False),
    scratch_types=[
        pltpu.VMEM((BLKE,), _i32),
        pltpu.VMEM((BLKE,), _i32),
        pltpu.VMEM((BLKE,), _f32),
        pltpu.VMEM((K,), _i32),
        pltpu.VMEM((K,), _i32),
        pltpu.VMEM((K, CH), _f32),
        pltpu.VMEM((K, CH), _f32),
        pltpu.VMEM((NSEG,), _f32),
        pltpu.VMEM((NW, MBLK), _f32),
        pltpu.VMEM_SHARED((NSEG, CH), _f32),
        pltpu.SemaphoreType.DMA,
        pltpu.SemaphoreType.DMA,
    ],
)


def _scu_body(x_hbm, src_hbm, dst_hbm, w_hbm,
              upart_hbm,
              idxsblk, idxdblk, wblk, srcb0, srcb1, xrows0, xrows1,
              spmem, st0, st1):
    c = lax.axis_index("c")
    sid = lax.axis_index("s")
    w = _worker_id()
    base0 = w * _c(EPW)
    lanes = lax.iota(_i32, 16)

    _zero_rows_buf(xrows0)
    _zero_spmem_slice(spmem, xrows0, sid)
    plsc.subcore_barrier()

    xrows = (xrows0, xrows1)
    srcb = (srcb0, srcb1)
    st = (st0, st1)

    def block(b, carry):
        eb0 = base0 + b * _c(BLKE)
        pltpu.sync_copy(src_hbm.at[pl.ds(eb0, BLKE)], idxsblk)
        pltpu.sync_copy(dst_hbm.at[pl.ds(eb0, BLKE)], idxdblk)
        pltpu.sync_copy(w_hbm.at[pl.ds(eb0, BLKE)], wblk)
        ds_ = [None] * BLK
        ds_[0] = pltpu.async_copy(x_hbm.at[idxdblk.at[pl.ds(0, K)]],
                                  xrows0, st0)
        for j in range(BLK):
            jm = j % 2
            if j + 1 < BLK:
                ds_[j + 1] = pltpu.async_copy(
                    x_hbm.at[idxdblk.at[pl.ds((j + 1) * K, K)]],
                    xrows[(j + 1) % 2], st[(j + 1) % 2])
            ds_[j].wait()
            for g in range(NG):
                sv = idxsblk[pl.ds(j * K + g * 16, 16)]
                srcb[jm][pl.ds(g * 16, 16)] = sv
                ev = wblk[pl.ds(j * K + g * 16, 16)]
                eidx = lanes + _c(g * 16)

                def sbody(i2, carry2, jm=jm, eidx=eidx, ev=ev):
                    ch0 = i2 * _c(8)
                    for k in range(8):
                        col = jnp.zeros((16,), _i32) + (ch0 + _c(k))
                        tv = plsc.load_gather(xrows[jm], [eidx, col])
                        plsc.store_scatter(xrows[jm], [eidx, col], tv * ev)
                    return carry2
                lax.fori_loop(_c(0), _c(CH // 8), sbody, _c(0))
            pltpu.sync_copy(xrows[jm], spmem.at[srcb[jm]], add=True)
        return carry
    lax.fori_loop(_c(0), _c(NBLOCK), block, _c(0))

    plsc.subcore_barrier()
    _dump_spmem(spmem, xrows0, upart_hbm, c, sid)


_scu = pl.kernel(
    _scu_body,
    out_type=[jax.ShapeDtypeStruct((NC, NSEG, CH), _f32)],
    mesh=_MESH,
    compiler_params=pltpu.CompilerParams(needs_layout_passes=False),
    scratch_types=[
        pltpu.VMEM((BLKE,), _i32),
        pltpu.VMEM((BLKE,), _i32),
        pltpu.VMEM((BLKE,), _f32),
        pltpu.VMEM((K,), _i32),
        pltpu.VMEM((K,), _i32),
        pltpu.VMEM((K, CH), _f32),
        pltpu.VMEM((K, CH), _f32),
        pltpu.VMEM_SHARED((NSEG, CH), _f32),
        pltpu.SemaphoreType.DMA,
        pltpu.SemaphoreType.DMA,
    ],
)


_RB = 1280


def _z(v=0):
    return jnp.array(v, _i32)


def _tc_prep_body(x_ref, rel_ref, a_ref):
    r = pl.program_id(1)
    a_ref[...] = x_ref[...] * rel_ref[pl.ds(r, 1), :]


_tc_prep = pl.pallas_call(
    _tc_prep_body,
    grid=(NSEG // _RB, NRELROW),
    in_specs=[pl.BlockSpec((_RB, CH), lambda b, r: (b, _z())),
              pl.BlockSpec((NRELROW, CH), lambda b, r: (_z(), _z())),],
    out_specs=pl.BlockSpec((_RB, CH), lambda b, r: (r * _z(NSEG // _RB) + b, _z())),
    out_shape=jax.ShapeDtypeStruct((NRELROW * NSEG, CH), _f32),
)


def _norm_rows(a):
    ss = jnp.sum(a * a, axis=1, keepdims=True)
    return a * lax.rsqrt(jnp.maximum(ss, 1e-24))


def _tc_merge_prep_body(pp_ref, rel_ref, x_ref, a_ref):
    r = pl.program_id(1)
    y = _norm_rows(pp_ref[0] + pp_ref[1])
    x_ref[...] = y
    a_ref[...] = y * rel_ref[pl.ds(r, 1), :]


_tc_merge_prep = pl.pallas_call(
    _tc_merge_prep_body,
    grid=(NSEG // _RB, NRELROW),
    in_specs=[pl.BlockSpec((NC, _RB, CH), lambda b, r: (_z(), b, _z())),
              pl.BlockSpec((NRELROW, CH), lambda b, r: (_z(), _z())),],
    out_specs=[pl.BlockSpec((_RB, CH), lambda b, r: (b, _z())),
               pl.BlockSpec((_RB, CH), lambda b, r: (r * _z(NSEG // _RB) + b, _z()))],
    out_shape=[jax.ShapeDtypeStruct((NSEG, CH), _f32),
               jax.ShapeDtypeStruct((NRELROW * NSEG, CH), _f32)],
)


def _tc_merge_body(pp_ref, x_ref):
    x_ref[...] = _norm_rows(pp_ref[0] + pp_ref[1])


_tc_merge = pl.pallas_call(
    _tc_merge_body,
    grid=(NSEG // _RB,),
    in_specs=[pl.BlockSpec((NC, _RB, CH), lambda b: (_z(), b, _z()))],
    out_specs=pl.BlockSpec((_RB, CH), lambda b: (b, _z())),
    out_shape=jax.ShapeDtypeStruct((NSEG, CH), _f32),
)


def kernel(user_emb, item_emb, edge_index, edge_type, inter_edge,
           inter_edge_w, relation_emb):
    del user_emb
    head = edge_index[0].astype(_i32)
    tail = edge_index[1].astype(_i32)
    rel = edge_type.astype(_i32)
    src = inter_edge[0].astype(_i32)
    dst = inter_edge[1].astype(_i32)
    iw = inter_edge_w.astype(_f32)
    relemb = relation_emb.astype(_f32)

    x = jnp.pad(item_emb.astype(_f32), ((0, NSEG - NENT), (0, 0)))
    a = _tc_prep(x, relemb)
    for hop in range(2):
        scores, maxpart = _sca(a, x, head, rel, tail)
        (ypart,) = _scb(x, head, tail, scores, maxpart)
        if hop == 0:
            x, a = _tc_merge_prep(ypart, relemb)
        else:
            x = _tc_merge(ypart)
    (upart,) = _scu(x, src, dst, iw)
    user_out = _tc_merge(upart)
    return user_out[:NENT], x[:NENT]


# speedup vs baseline: 1.2296x; 1.0054x over previous
import jax
import jax.numpy as jnp
from jax import lax
from jax.experimental import pallas as pl
from jax.experimental.pallas import tpu as pltpu
from jax.experimental.pallas import tpu_sc as plsc

NENT = 10000
NSEG = 10240
CH = 128
NEDGE = 320000
NRELROW = 9
NC = 2
NS = 16
NW = NC * NS
EPW = NEDGE // NW
K = 80
NCHUNK = EPW // K
NG = K // 16
BLK = 5
BLKE = BLK * K
NBLOCK = NCHUNK // BLK
ROWS_PER_TILE = NSEG // NS
MBLK = 128
MROUNDS = NSEG // MBLK

_MESH = plsc.VectorSubcoreMesh(core_axis_name="c", subcore_axis_name="s")
_f32 = jnp.float32
_i32 = jnp.int32


def _c(v):
    return jnp.array(v, _i32)


def _worker_id():
    return lax.axis_index("s") * _c(NC) + lax.axis_index("c")


def _retry_scatter_max(tab, hidx, sv):
    def bdy(go):
        cur = plsc.load_gather(tab, [hidx])
        plsc.store_scatter(tab, [hidx], jnp.maximum(sv, cur), mask=sv > cur)
        chk = plsc.load_gather(tab, [hidx])
        return jnp.max((sv > chk).astype(_i32))
    lax.while_loop(lambda go: go > _c(0), bdy, _c(1))


def _sca_body(a_hbm, x_hbm, head_hbm, rel_hbm, tail_hbm,
              scores_hbm, maxpart_hbm,
              headall, relall, tailall, cidxall, sball, maxtab,
              arows0, arows1, trows0, trows1, cidxb0, cidxb1, tailb0, tailb1,
              sa0, sa1, st0, st1):
    w = _worker_id()
    base0 = w * _c(EPW)
    lanes = lax.iota(_i32, 16)
    zero16 = jnp.zeros((16,), _f32)
    zero16i = jnp.zeros((16,), _i32)

    pltpu.sync_copy(head_hbm.at[pl.ds(base0, EPW)], headall)
    pltpu.sync_copy(rel_hbm.at[pl.ds(base0, EPW)], relall)
    pltpu.sync_copy(tail_hbm.at[pl.ds(base0, EPW)], tailall)

    def initb(i, carry):
        plsc.store_scatter(maxtab, [lanes + i * _c(16)], zero16)
        return carry
    lax.fori_loop(_c(0), _c(NSEG // 16), initb, _c(0))

    def cidx_build(i, carry):
        idx = lanes + i * _c(16)
        h16 = plsc.load_gather(headall, [idx])
        r16 = plsc.load_gather(relall, [idx])
        rid = jnp.where(r16 == _c(0), _c(NRELROW - 1), r16 - _c(1))
        plsc.store_scatter(cidxall, [idx], rid * _c(NSEG) + h16)
        return carry
    lax.fori_loop(_c(0), _c(EPW // 16), cidx_build, _c(0))

    arows = (arows0, arows1)
    trows = (trows0, trows1)
    cidxb = (cidxb0, cidxb1)
    tailb = (tailb0, tailb1)
    sa = (sa0, sa1)
    st = (st0, st1)

    def issue(ci, jmod):
        base_l = ci * _c(K)
        for g in range(NG):
            gidx = lanes + base_l + _c(g * 16)
            cidxb[jmod][pl.ds(g * 16, 16)] = plsc.load_gather(cidxall, [gidx])
            tailb[jmod][pl.ds(g * 16, 16)] = plsc.load_gather(tailall, [gidx])
        da = pltpu.async_copy(a_hbm.at[cidxb[jmod]], arows[jmod], sa[jmod])
        dt = pltpu.async_copy(x_hbm.at[tailb[jmod]], trows[jmod], st[jmod])
        return da, dt

    def block(b, carry):
        ci0 = b * _c(BLK)
        ds_ = [None] * BLK
        ds_[0] = issue(ci0, 0)
        for j in range(BLK):
            jm = j % 2
            if j + 1 < BLK:
                ds_[j + 1] = issue(ci0 + _c(j + 1), (j + 1) % 2)
            ds_[j][0].wait()
            ds_[j][1].wait()
            base_l = (ci0 + _c(j)) * _c(K)
            for g in range(NG):
                eidx = lanes + _c(g * 16)

                def cbody(i2, accs, jm=jm, eidx=eidx):
                    ch0 = i2 * _c(8)
                    outs = list(accs)
                    for k in range(8):
                        col = zero16i + (ch0 + _c(k))
                        av = plsc.load_gather(arows[jm], [eidx, col])
                        tv = plsc.load_gather(trows[jm], [eidx, col])
                        outs[k % 4] = outs[k % 4] + av * tv
                    return tuple(outs)
                accs = lax.fori_loop(_c(0), _c(CH // 8), cbody,
                                     (zero16, zero16, zero16, zero16))
                dot = (accs[0] + accs[1]) + (accs[2] + accs[3])
                sv = jnp.exp(dot)
                gidx = lanes + base_l + _c(g * 16)
                plsc.store_scatter(sball, [gidx], sv)
                hidx = plsc.load_gather(headall, [gidx])
                _retry_scatter_max(maxtab, hidx, sv)
        return carry
    lax.fori_loop(_c(0), _c(NBLOCK), block, _c(0))

    pltpu.sync_copy(sball, scores_hbm.at[pl.ds(base0, EPW)])
    pltpu.sync_copy(maxtab, maxpart_hbm.at[w])


_sca = pl.kernel(
    _sca_body,
    out_type=[jax.ShapeDtypeStruct((NEDGE,), _f32),
              jax.ShapeDtypeStruct((NW, NSEG), _f32)],
    mesh=_MESH,
    compiler_params=pltpu.CompilerParams(needs_layout_passes=False),
    scratch_types=[
        pltpu.VMEM((EPW,), _i32),
        pltpu.VMEM((EPW,), _i32),
        pltpu.VMEM((EPW,), _i32),
        pltpu.VMEM((EPW,), _i32),
        pltpu.VMEM((EPW,), _f32),
        pltpu.VMEM((NSEG,), _f32),
        pltpu.VMEM((K, CH), _f32),
        pltpu.VMEM((K, CH), _f32),
        pltpu.VMEM((K, CH), _f32),
        pltpu.VMEM((K, CH), _f32),
        pltpu.VMEM((K,), _i32),
        pltpu.VMEM((K,), _i32),
        pltpu.VMEM((K,), _i32),
        pltpu.VMEM((K,), _i32),
        pltpu.SemaphoreType.DMA,
        pltpu.SemaphoreType.DMA,
        pltpu.SemaphoreType.DMA,
        pltpu.SemaphoreType.DMA,
    ],
)


def _zero_rows_buf(buf):
    lanes = lax.iota(_i32, 16)
    zero16 = jnp.zeros((16,), _f32)

    def zb(i, carry):
        flat = lanes + i * _c(16)
        plsc.store_scatter(buf, [lax.shift_right_logical(flat, _c(7)),
                                 lax.bitwise_and(flat, _c(127))], zero16)
        return carry
    lax.fori_loop(_c(0), _c(K * CH // 16), zb, _c(0))


def _zero_spmem_slice(spmem, buf, sid):
    def zloop(j, carry):
        pltpu.sync_copy(
            buf, spmem.at[pl.ds(sid * _c(ROWS_PER_TILE) + j * _c(K), K)])
        return carry
    lax.fori_loop(_c(0), _c(ROWS_PER_TILE // K), zloop, _c(0))


def _dump_spmem(spmem, buf, ypart_hbm, c, sid):
    for j in range(ROWS_PER_TILE // K):
        row = sid * _c(ROWS_PER_TILE) + _c(j * K)
        pltpu.sync_copy(spmem.at[pl.ds(row, K)], buf)
        pltpu.sync_copy(buf, ypart_hbm.at[c, pl.ds(row, K)])


def _scb_body(x_hbm, head_hbm, tail_hbm, scores_hbm, maxpart_hbm,
              ypart_hbm,
              idxhblk, idxtblk, sblk, headb0, headb1, trows0, trows1,
              mtab, mstage, spmem, st0, st1):
    c = lax.axis_index("c")
    sid = lax.axis_index("s")
    w = _worker_id()
    base0 = w * _c(EPW)
    lanes = lax.iota(_i32, 16)

    _zero_rows_buf(trows0)
    _zero_spmem_slice(spmem, trows0, sid)
    plsc.subcore_barrier()

    def mround(r, carry):
        seg0 = r * _c(MBLK)
        pltpu.sync_copy(maxpart_hbm.at[:, pl.ds(seg0, MBLK)], mstage)
        for g in range(MBLK // 16):
            idx = lanes + _c(g * 16)
            m = plsc.load_gather(mstage, [jnp.full((16,), 0, _i32), idx])
            for j in range(1, NW):
                vj = plsc.load_gather(mstage, [jnp.full((16,), j, _i32), idx])
                m = jnp.maximum(m, vj)
            plsc.store_scatter(mtab, [idx + seg0], m)
        return carry
    lax.fori_loop(_c(0), _c(MROUNDS), mround, _c(0))

    trows = (trows0, trows1)
    headb = (headb0, headb1)
    st = (st0, st1)

    def block(b, carry):
        eb0 = base0 + b * _c(BLKE)
        pltpu.sync_copy(head_hbm.at[pl.ds(eb0, BLKE)], idxhblk)
        pltpu.sync_copy(tail_hbm.at[pl.ds(eb0, BLKE)], idxtblk)
        pltpu.sync_copy(scores_hbm.at[pl.ds(eb0, BLKE)], sblk)
        ds_ = [None] * BLK
        ds_[0] = pltpu.async_copy(x_hbm.at[idxtblk.at[pl.ds(0, K)]],
                                  trows0, st0)
        for j in range(BLK):
            jm = j % 2
            if j + 1 < BLK:
                ds_[j + 1] = pltpu.async_copy(
                    x_hbm.at[idxtblk.at[pl.ds((j + 1) * K, K)]],
                    trows[(j + 1) % 2], st[(j + 1) % 2])
            ds_[j].wait()
            for g in range(NG):
                hv = idxhblk[pl.ds(j * K + g * 16, 16)]
                headb[jm][pl.ds(g * 16, 16)] = hv
                m16 = plsc.load_gather(mtab, [hv])
                ev = jnp.exp(sblk[pl.ds(j * K + g * 16, 16)] - m16)
                eidx = lanes + _c(g * 16)

                def sbody(i2, carry2, jm=jm, eidx=eidx, ev=ev):
                    ch0 = i2 * _c(8)
                    for k in range(8):
                        col = jnp.zeros((16,), _i32) + (ch0 + _c(k))
                        tv = plsc.load_gather(trows[jm], [eidx, col])
                        plsc.store_scatter(trows[jm], [eidx, col], tv * ev)
                    return carry2
                lax.fori_loop(_c(0), _c(CH // 8), sbody, _c(0))
            pltpu.sync_copy(trows[jm], spmem.at[headb[jm]], add=True)
        return carry
    lax.fori_loop(_c(0), _c(NBLOCK), block, _c(0))

    plsc.subcore_barrier()
    _dump_spmem(spmem, trows0, ypart_hbm, c, sid)


_scb = pl.kernel(
    _scb_body,
    out_type=[jax.ShapeDtypeStruct((NC, NSEG, CH), _f32)],
    mesh=_MESH,
    compiler_params=pltpu.CompilerParams(needs_layout_passes=False),
    scratch_types=[
        pltpu.VMEM((BLKE,), _i32),
        pltpu.VMEM((BLKE,), _i32),
        pltpu.VMEM((BLKE,), _f32),
        pltpu.VMEM((K,), _i32),
        pltpu.VMEM((K,), _i32),
        pltpu.VMEM((K, CH), _f32),
        pltpu.VMEM((K, CH), _f32),
        pltpu.VMEM((NSEG,), _f32),
        pltpu.VMEM((NW, MBLK), _f32),
        pltpu.VMEM_SHARED((NSEG, CH), _f32),
        pltpu.SemaphoreType.DMA,
        pltpu.SemaphoreType.DMA,
    ],
)


def _scu_body(x_hbm, src_hbm, dst_hbm, w_hbm,
              upart_hbm,
              idxsblk, idxdblk, wblk, srcb0, srcb1, xrows0, xrows1,
              orows0, orows1, spmem, st0, st1, ss0, ss1):
    c = lax.axis_index("c")
    sid = lax.axis_index("s")
    w = _worker_id()
    base0 = w * _c(EPW)
    lanes = lax.iota(_i32, 16)

    _zero_rows_buf(xrows0)
    _zero_spmem_slice(spmem, xrows0, sid)
    plsc.subcore_barrier()

    xrows = (xrows0, xrows1)
    srcb = (srcb0, srcb1)
    orows = (orows0, orows1)
    st = (st0, st1)
    ss = (ss0, ss1)

    def block(b, carry):
        eb0 = base0 + b * _c(BLKE)
        pltpu.sync_copy(src_hbm.at[pl.ds(eb0, BLKE)], idxsblk)
        pltpu.sync_copy(dst_hbm.at[pl.ds(eb0, BLKE)], idxdblk)
        pltpu.sync_copy(w_hbm.at[pl.ds(eb0, BLKE)], wblk)
        ds_ = [None] * BLK
        ds_[0] = pltpu.async_copy(x_hbm.at[idxdblk.at[pl.ds(0, K)]],
                                  xrows0, st0)
        for j in range(BLK):
            jm = j % 2
            if j + 1 < BLK:
                ds_[j + 1] = pltpu.async_copy(
                    x_hbm.at[idxdblk.at[pl.ds((j + 1) * K, K)]],
                    xrows[(j + 1) % 2], st[(j + 1) % 2])
            ds_[j].wait()
            if j >= 2:
                pltpu.make_async_copy(x_hbm.at[pl.ds(0, K)],
                                      orows[jm], ss[jm]).wait()
            for g in range(NG):
                sv = idxsblk[pl.ds(j * K + g * 16, 16)]
                srcb[jm][pl.ds(g * 16, 16)] = sv
                ev = wblk[pl.ds(j * K + g * 16, 16)]
                eidx = lanes + _c(g * 16)

                def sbody(i2, carry2, jm=jm, eidx=eidx, ev=ev):
                    ch0 = i2 * _c(8)
                    for k in range(8):
                        col = jnp.zeros((16,), _i32) + (ch0 + _c(k))
                        tv = plsc.load_gather(xrows[jm], [eidx, col])
                        plsc.store_scatter(orows[jm], [eidx, col], tv * ev)
                    return carry2
                lax.fori_loop(_c(0), _c(CH // 8), sbody, _c(0))
            pltpu.async_copy(orows[jm], spmem.at[srcb[jm]], ss[jm], add=True)
        pltpu.make_async_copy(x_hbm.at[pl.ds(0, K)], orows[1], ss[1]).wait()
        pltpu.make_async_copy(x_hbm.at[pl.ds(0, K)], orows[0], ss[0]).wait()
        return carry
    lax.fori_loop(_c(0), _c(NBLOCK), block, _c(0))

    plsc.subcore_barrier()
    _dump_spmem(spmem, xrows0, upart_hbm, c, sid)


_scu = pl.kernel(
    _scu_body,
    out_type=[jax.ShapeDtypeStruct((NC, NSEG, CH), _f32)],
    mesh=_MESH,
    compiler_params=pltpu.CompilerParams(needs_layout_passes=False),
    scratch_types=[
        pltpu.VMEM((BLKE,), _i32),
        pltpu.VMEM((BLKE,), _i32),
        pltpu.VMEM((BLKE,), _f32),
        pltpu.VMEM((K,), _i32),
        pltpu.VMEM((K,), _i32),
        pltpu.VMEM((K, CH), _f32),
        pltpu.VMEM((K, CH), _f32),
        pltpu.VMEM((K, CH), _f32),
        pltpu.VMEM((K, CH), _f32),
        pltpu.VMEM_SHARED((NSEG, CH), _f32),
        pltpu.SemaphoreType.DMA,
        pltpu.SemaphoreType.DMA,
        pltpu.SemaphoreType.DMA,
        pltpu.SemaphoreType.DMA,
    ],
)


_RB = 1280


def _z(v=0):
    return jnp.array(v, _i32)


def _tc_prep_body(x_ref, rel_ref, a_ref):
    r = pl.program_id(1)
    a_ref[...] = x_ref[...] * rel_ref[pl.ds(r, 1), :]


_tc_prep = pl.pallas_call(
    _tc_prep_body,
    grid=(NSEG // _RB, NRELROW),
    in_specs=[pl.BlockSpec((_RB, CH), lambda b, r: (b, _z())),
              pl.BlockSpec((NRELROW, CH), lambda b, r: (_z(), _z())),],
    out_specs=pl.BlockSpec((_RB, CH), lambda b, r: (r * _z(NSEG // _RB) + b, _z())),
    out_shape=jax.ShapeDtypeStruct((NRELROW * NSEG, CH), _f32),
)


def _norm_rows(a):
    ss = jnp.sum(a * a, axis=1, keepdims=True)
    return a * lax.rsqrt(jnp.maximum(ss, 1e-24))


def _tc_merge_prep_body(pp_ref, rel_ref, x_ref, a_ref):
    r = pl.program_id(1)
    y = _norm_rows(pp_ref[0] + pp_ref[1])
    x_ref[...] = y
    a_ref[...] = y * rel_ref[pl.ds(r, 1), :]


_tc_merge_prep = pl.pallas_call(
    _tc_merge_prep_body,
    grid=(NSEG // _RB, NRELROW),
    in_specs=[pl.BlockSpec((NC, _RB, CH), lambda b, r: (_z(), b, _z())),
              pl.BlockSpec((NRELROW, CH), lambda b, r: (_z(), _z())),],
    out_specs=[pl.BlockSpec((_RB, CH), lambda b, r: (b, _z())),
               pl.BlockSpec((_RB, CH), lambda b, r: (r * _z(NSEG // _RB) + b, _z()))],
    out_shape=[jax.ShapeDtypeStruct((NSEG, CH), _f32),
               jax.ShapeDtypeStruct((NRELROW * NSEG, CH), _f32)],
)


def _tc_merge_body(pp_ref, x_ref):
    x_ref[...] = _norm_rows(pp_ref[0] + pp_ref[1])


_tc_merge = pl.pallas_call(
    _tc_merge_body,
    grid=(NSEG // _RB,),
    in_specs=[pl.BlockSpec((NC, _RB, CH), lambda b: (_z(), b, _z()))],
    out_specs=pl.BlockSpec((_RB, CH), lambda b: (b, _z())),
    out_shape=jax.ShapeDtypeStruct((NSEG, CH), _f32),
)


def kernel(user_emb, item_emb, edge_index, edge_type, inter_edge,
           inter_edge_w, relation_emb):
    del user_emb
    head = edge_index[0].astype(_i32)
    tail = edge_index[1].astype(_i32)
    rel = edge_type.astype(_i32)
    src = inter_edge[0].astype(_i32)
    dst = inter_edge[1].astype(_i32)
    iw = inter_edge_w.astype(_f32)
    relemb = relation_emb.astype(_f32)

    x = jnp.pad(item_emb.astype(_f32), ((0, NSEG - NENT), (0, 0)))
    a = _tc_prep(x, relemb)
    for hop in range(2):
        scores, maxpart = _sca(a, x, head, rel, tail)
        (ypart,) = _scb(x, head, tail, scores, maxpart)
        if hop == 0:
            x, a = _tc_merge_prep(ypart, relemb)
        else:
            x = _tc_merge(ypart)
    (upart,) = _scu(x, src, dst, iw)
    user_out = _tc_merge(upart)
    return user_out[:NENT], x[:NENT]
